# baseline (speedup 1.0000x reference)
import jax
import jax.numpy as jnp
from jax.experimental import pallas as pl
from jax.experimental.pallas import tpu as pltpu

_BN = 262144


def _mlp_body(x_ref, w1_ref, b1_ref, w2_ref, b2_ref, la_ref, lb_ref):
    x = x_ref[...]
    z = jnp.dot(w1_ref[...], x, preferred_element_type=jnp.float32)
    z = z + b1_ref[...]
    h = jnp.maximum(z, 0.01 * z)
    o = jnp.dot(w2_ref[...], h,
                preferred_element_type=jnp.float32)
    la_ref[...] = o[0, :] + b2_ref[0]
    lb_ref[...] = o[1, :] + b2_ref[1]


@jax.jit
def kernel(X, W1, b1, W2, b2):
    n, d = X.shape
    xt = X.T

    w1t = W1.T
    b1c = b1.reshape(d, 1)
    w2t = jnp.concatenate(
        [W2.T, jnp.zeros((8 - W2.shape[1], d), W2.dtype)], axis=0
    )

    grid = (pl.cdiv(n, _BN),)
    la, lb = pl.pallas_call(
        _mlp_body,
        grid=grid,
        in_specs=[
            pl.BlockSpec((d, _BN), lambda j: (0, j)),
            pl.BlockSpec((d, d), lambda j: (0, 0)),
            pl.BlockSpec((d, 1), lambda j: (0, 0)),
            pl.BlockSpec((8, d), lambda j: (0, 0)),
            pl.BlockSpec(memory_space=pltpu.MemorySpace.SMEM),
        ],
        out_specs=[
            pl.BlockSpec((_BN,), lambda j: (j,)),
            pl.BlockSpec((_BN,), lambda j: (j,)),
        ],
        out_shape=[
            jax.ShapeDtypeStruct((n,), X.dtype),
            jax.ShapeDtypeStruct((n,), X.dtype),
        ],
        compiler_params=pltpu.CompilerParams(
            dimension_semantics=("arbitrary",),
        ),
    )(xt, w1t, b1c, w2t, b2)
    return la, lb


# speedup vs baseline: 1.0977x; 1.0977x over previous
import jax
import jax.numpy as jnp
from jax.experimental import pallas as pl
from jax.experimental.pallas import tpu as pltpu

_BN = 262144


def _mlp_body(x_ref, w1_ref, b1_ref, w2_ref, b2_ref, la_ref, lb_ref):
    x = x_ref[...]
    z = jnp.dot(w1_ref[...], x, preferred_element_type=jnp.float32)
    z = z + b1_ref[...]
    h = jnp.maximum(z, 0.01 * z)
    o = jnp.dot(w2_ref[...], h,
                preferred_element_type=jnp.float32)
    o = o + b2_ref[...]
    la_ref[...] = o[0, :]
    lb_ref[...] = o[1, :]


@jax.jit
def kernel(X, W1, b1, W2, b2):
    n, d = X.shape
    xt = X.T

    w1t = W1.T
    b1c = b1.reshape(d, 1)
    w2t = jnp.concatenate(
        [W2.T, jnp.zeros((8 - W2.shape[1], d), W2.dtype)], axis=0
    )
    b2c = jnp.concatenate(
        [b2, jnp.zeros((8 - b2.shape[0],), b2.dtype)]
    ).reshape(8, 1)

    grid = (pl.cdiv(n, _BN),)
    la, lb = pl.pallas_call(
        _mlp_body,
        grid=grid,
        in_specs=[
            pl.BlockSpec((d, _BN), lambda j: (0, j)),
            pl.BlockSpec((d, d), lambda j: (0, 0)),
            pl.BlockSpec((d, 1), lambda j: (0, 0)),
            pl.BlockSpec((8, d), lambda j: (0, 0)),
            pl.BlockSpec((8, 1), lambda j: (0, 0)),
        ],
        out_specs=[
            pl.BlockSpec((_BN,), lambda j: (j,)),
            pl.BlockSpec((_BN,), lambda j: (j,)),
        ],
        out_shape=[
            jax.ShapeDtypeStruct((n,), X.dtype),
            jax.ShapeDtypeStruct((n,), X.dtype),
        ],
        compiler_params=pltpu.CompilerParams(
            dimension_semantics=("parallel",),
        ),
    )(xt, w1t, b1c, w2t, b2c)
    return la, lb


# speedup vs baseline: 1.0982x; 1.0005x over previous
import jax
import jax.numpy as jnp
from jax.experimental import pallas as pl
from jax.experimental.pallas import tpu as pltpu

_BN = 262144


def _mlp_body(x_ref, w1_ref, b1_ref, w2_ref, b2_ref, la_ref, lb_ref):
    x = x_ref[...]
    z = jnp.dot(w1_ref[...], x, preferred_element_type=jnp.float32)
    z = z + b1_ref[...]
    h = jnp.maximum(z, 0.01 * z)
    o = jnp.dot(w2_ref[...], h.astype(jnp.bfloat16),
                preferred_element_type=jnp.float32)
    o = o + b2_ref[...]
    la_ref[...] = o[0, :]
    lb_ref[...] = o[1, :]


@jax.jit
def kernel(X, W1, b1, W2, b2):
    n, d = X.shape
    xt = X.T

    w1t = W1.T
    b1c = b1.reshape(d, 1)
    w2t = jnp.concatenate(
        [W2.T, jnp.zeros((8 - W2.shape[1], d), W2.dtype)], axis=0
    )
    b2c = jnp.concatenate(
        [b2, jnp.zeros((8 - b2.shape[0],), b2.dtype)]
    ).reshape(8, 1)

    grid = (pl.cdiv(n, _BN),)
    la, lb = pl.pallas_call(
        _mlp_body,
        grid=grid,
        in_specs=[
            pl.BlockSpec((d, _BN), lambda j: (0, j)),
            pl.BlockSpec((d, d), lambda j: (0, 0)),
            pl.BlockSpec((d, 1), lambda j: (0, 0)),
            pl.BlockSpec((8, d), lambda j: (0, 0)),
            pl.BlockSpec((8, 1), lambda j: (0, 0)),
        ],
        out_specs=[
            pl.BlockSpec((_BN,), lambda j: (j,)),
            pl.BlockSpec((_BN,), lambda j: (j,)),
        ],
        out_shape=[
            jax.ShapeDtypeStruct((n,), X.dtype),
            jax.ShapeDtypeStruct((n,), X.dtype),
        ],
        compiler_params=pltpu.CompilerParams(
            dimension_semantics=("parallel",),
        ),
    )(xt, w1t, b1c, w2t, b2c)
    return la, lb
